# Initial kernel scaffold; baseline (speedup 1.0000x reference)
import jax
import jax.numpy as jnp
from jax.experimental import pallas as pl


def kernel(x, edge_index, W1, b1, g1, be1, W2, b2, g2, be2, W3, b3, g3, be3, W4, b4, g4, be4, ln_g, ln_b):
    raise NotImplementedError("write your pallas kernel here")


# speedup vs baseline: 6.2786x; 6.2786x over previous
import functools

import jax
import jax.numpy as jnp
from jax import lax
from jax.experimental import pallas as pl
from jax.experimental.pallas import tpu as pltpu
from jax.experimental.pallas import tpu_sc as plsc

N = 10000
E = 320000
D = 128
CHUNK = 128
NCORES = 2
NSUB = 16
NW = NCORES * NSUB
CPT = 80
NPAD = NW * CPT
RPT = 632
NACC = NSUB * RPT

_MESH = plsc.VectorSubcoreMesh(core_axis_name="c", subcore_axis_name="s")


@functools.partial(
    pl.kernel,
    out_type=jax.ShapeDtypeStruct((NCORES, NACC, D), jnp.float32),
    mesh=_MESH,
    scratch_types=[
        pltpu.VMEM((CPT, CHUNK), jnp.int32),
        pltpu.VMEM((CHUNK, D), jnp.float32),
        pltpu.VMEM_SHARED((NACC, D), jnp.float32),
    ],
)
def _sc_degree(dst_hbm, ones_hbm, zeros_hbm, out_hbm, dst_v, ones_v, acc_sh):
    cid = lax.axis_index("c")
    sid = lax.axis_index("s")
    wid = sid * NCORES + cid
    pltpu.sync_copy(zeros_hbm.at[pl.ds(sid * RPT, RPT)],
                    acc_sh.at[pl.ds(sid * RPT, RPT)])
    pltpu.sync_copy(ones_hbm, ones_v)
    pltpu.sync_copy(dst_hbm.at[pl.ds(wid * CPT, CPT)], dst_v)
    plsc.subcore_barrier()

    def body(j, carry):
        pltpu.sync_copy(ones_v, acc_sh.at[dst_v.at[j]], add=True)
        return carry

    lax.fori_loop(0, CPT, body, 0)
    plsc.subcore_barrier()
    pltpu.sync_copy(acc_sh.at[pl.ds(sid * RPT, RPT)],
                    out_hbm.at[cid, pl.ds(sid * RPT, RPT)])


@functools.partial(
    pl.kernel,
    out_type=jax.ShapeDtypeStruct((NCORES, NACC, D), jnp.float32),
    mesh=_MESH,
    scratch_types=[
        pltpu.VMEM((CPT, CHUNK), jnp.int32),
        pltpu.VMEM((CPT, CHUNK), jnp.int32),
        pltpu.VMEM((CHUNK, D), jnp.float32),
        pltpu.VMEM_SHARED((NACC, D), jnp.float32),
        pltpu.SemaphoreType.DMA,
    ],
)
def _sc_scatter(hs_hbm, src_hbm, dst_hbm, zeros_hbm, out_hbm,
                src_v, dst_v, rows_v, acc_sh, sem):
    cid = lax.axis_index("c")
    sid = lax.axis_index("s")
    wid = sid * NCORES + cid
    pltpu.sync_copy(zeros_hbm.at[pl.ds(sid * RPT, RPT)],
                    acc_sh.at[pl.ds(sid * RPT, RPT)])
    pltpu.sync_copy(src_hbm.at[pl.ds(wid * CPT, CPT)], src_v)
    pltpu.sync_copy(dst_hbm.at[pl.ds(wid * CPT, CPT)], dst_v)
    plsc.subcore_barrier()

    def body(j, carry):
        pltpu.async_copy(hs_hbm.at[src_v.at[j]], rows_v, sem).wait()
        pltpu.sync_copy(rows_v, acc_sh.at[dst_v.at[j]], add=True)
        return carry

    lax.fori_loop(0, CPT, body, 0)
    plsc.subcore_barrier()
    pltpu.sync_copy(acc_sh.at[pl.ds(sid * RPT, RPT)],
                    out_hbm.at[cid, pl.ds(sid * RPT, RPT)])


def _dinv(deg_ref):
    d = deg_ref[0, :N, 0:1] + deg_ref[1, :N, 0:1] + 1.0
    return lax.rsqrt(d)


def _bn_lrelu(v, g, be):
    mu = jnp.mean(v, axis=0, keepdims=True)
    var = jnp.mean((v - mu) * (v - mu), axis=0, keepdims=True)
    o = g * (v - mu) * lax.rsqrt(var + 1e-5) + be
    return jnp.where(o > 0, o, 0.01 * o)


def _tc_pre_body(deg_ref, x_ref, w_ref, h_ref, hs_ref):
    dinv = _dinv(deg_ref)
    h = jnp.dot(x_ref[...], w_ref[...], preferred_element_type=jnp.float32)
    h_ref[...] = h
    hs_ref[...] = h * dinv


def _tc_mid_body(deg_ref, h_ref, s_ref, b_ref, g_ref, be_ref, w_ref,
                 hn_ref, hsn_ref):
    dinv = _dinv(deg_ref)
    s = s_ref[0, :N] + s_ref[1, :N]
    conv = dinv * s + (dinv * dinv) * h_ref[...] + b_ref[...]
    a = _bn_lrelu(conv, g_ref[...], be_ref[...])
    hn = jnp.dot(a, w_ref[...], preferred_element_type=jnp.float32)
    hn_ref[...] = hn
    hsn_ref[...] = hn * dinv


def _tc_fin_body(deg_ref, h_ref, s_ref, b_ref, x_ref, g_ref, be_ref,
                 lng_ref, lnb_ref, out_ref):
    dinv = _dinv(deg_ref)
    s = s_ref[0, :N] + s_ref[1, :N]
    conv = dinv * s + (dinv * dinv) * h_ref[...] + b_ref[...]
    v = conv + x_ref[...]
    mu = jnp.mean(v, axis=0, keepdims=True)
    var = jnp.mean((v - mu) * (v - mu), axis=0, keepdims=True)
    v = g_ref[...] * (v - mu) * lax.rsqrt(var + 1e-5) + be_ref[...]
    mu = jnp.mean(v, axis=1, keepdims=True)
    var = jnp.mean((v - mu) * (v - mu), axis=1, keepdims=True)
    out_ref[...] = lng_ref[...] * (v - mu) * lax.rsqrt(var + 1e-5) + lnb_ref[...]


_F32 = jnp.float32
_HH = [jax.ShapeDtypeStruct((N, D), _F32)] * 2

_tc_pre = pl.pallas_call(_tc_pre_body, out_shape=_HH)
_tc_mid = pl.pallas_call(_tc_mid_body, out_shape=_HH)
_tc_fin = pl.pallas_call(_tc_fin_body,
                         out_shape=jax.ShapeDtypeStruct((N, D), _F32))


def kernel(x, edge_index, W1, b1, g1, be1, W2, b2, g2, be2, W3, b3, g3, be3,
           W4, b4, g4, be4, ln_g, ln_b):
    pad = NPAD * CHUNK - E
    src = jnp.concatenate(
        [edge_index[0].astype(jnp.int32), jnp.zeros((pad,), jnp.int32)]
    ).reshape(NPAD, CHUNK)
    dst = jnp.concatenate(
        [edge_index[1].astype(jnp.int32), jnp.full((pad,), N, jnp.int32)]
    ).reshape(NPAD, CHUNK)
    zeros = jnp.zeros((NACC, D), _F32)
    deg = _sc_degree(dst, jnp.ones((CHUNK, D), _F32), zeros)

    h, hs = _tc_pre(deg, x, W1)
    for (bb, g, be, W) in ((b1, g1, be1, W2), (b2, g2, be2, W3),
                           (b3, g3, be3, W4)):
        s = _sc_scatter(hs, src, dst, zeros)
        h, hs = _tc_mid(deg, h, s, bb, g, be, W)
    s = _sc_scatter(hs, src, dst, zeros)
    return _tc_fin(deg, h, s, b4, x, g4, be4, ln_g, ln_b)
